# baseline (speedup 1.0000x reference)
import functools

import jax
import jax.numpy as jnp
import numpy as np
from jax import lax
from jax.experimental import pallas as pl
from jax.experimental.pallas import tpu as pltpu
from jax.experimental.pallas import tpu_sc as plsc

_EPS = 1e-5
_IDX_CHUNK = 128
_LANES = 16


def _rsqrt_vec(var, magic):
    seed = lax.bitcast_convert_type(
        magic - lax.shift_right_logical(
            lax.bitcast_convert_type(var, jnp.int32), 1),
        jnp.float32)
    y = seed
    for _ in range(3):
        y = y * (1.5 - 0.5 * var * y * y)
    return y


def _lane_sum(x, lane_iota):
    for k in (8, 4, 2, 1):
        perm = lax.bitwise_xor(lane_iota, jnp.int32(k))
        x = x + x.at[perm].get(mode="promise_in_bounds", unique_indices=True)
    return x


def _make_kernel(num_rows_total, table_rows, d_half, nc, ns):
    nw = nc * ns
    row_width = 2 * d_half
    n_sub = d_half // _LANES
    rows_per_w = num_rows_total // nw
    n_chunks = rows_per_w // _IDX_CHUNK
    stage_rows = 64
    n_groups = stage_rows // _LANES
    last_start = table_rows - stage_rows
    inv_n = jnp.float32(1.0 / d_half)
    mesh = plsc.VectorSubcoreMesh(core_axis_name="c", subcore_axis_name="s")

    @functools.partial(
        pl.kernel,
        mesh=mesh,
        out_type=jax.ShapeDtypeStruct(
            (nw * n_chunks, _IDX_CHUNK, row_width), jnp.float32),
        scratch_types=[
            pltpu.VMEM_SHARED((table_rows, row_width), jnp.float32),
            pltpu.VMEM((stage_rows * d_half,), jnp.float32),
            pltpu.VMEM((stage_rows, row_width), jnp.float32),
            pltpu.VMEM((_LANES,), jnp.int32),
            pltpu.VMEM((8 * d_half,), jnp.float32),
            pltpu.VMEM((d_half,), jnp.float32),
            pltpu.VMEM((d_half,), jnp.float32),
            pltpu.VMEM((d_half,), jnp.float32),
            pltpu.VMEM((d_half,), jnp.float32),
            pltpu.VMEM((n_chunks, _IDX_CHUNK), jnp.int32),
            pltpu.VMEM((n_chunks, _IDX_CHUNK, row_width), jnp.float32),
            pltpu.SemaphoreType.DMA((n_chunks,)),
            pltpu.SemaphoreType.DMA,
        ],
    )
    def k(depth_hbm, idx_hbm, tidx_hbm, dw_hbm, db_hbm, ttab_hbm, tw_hbm,
          tb_hbm, out_hbm, table_sp, raw_v, comb_v, tidx_v, ttab_v,
          dw_v, db_v, tw_v, tb_v, idx_v, rows_v, gsem, ssem):
        sid = lax.axis_index("s")
        wid = sid * nc + lax.axis_index("c")
        base = wid * n_chunks
        start = lax.min(sid * stage_rows, jnp.int32(last_start))

        pltpu.sync_copy(idx_hbm.at[pl.ds(base, n_chunks)], idx_v)
        pltpu.sync_copy(
            depth_hbm.at[pl.ds(start * d_half, stage_rows * d_half)], raw_v)
        pltpu.sync_copy(tidx_hbm, tidx_v)
        pltpu.sync_copy(dw_hbm, dw_v)
        pltpu.sync_copy(db_hbm, db_v)
        pltpu.sync_copy(tw_hbm, tw_v)
        pltpu.sync_copy(tb_hbm, tb_v)
        pltpu.sync_copy(ttab_hbm, ttab_v)

        lane_iota = lax.iota(jnp.int32, _LANES)
        zeros16 = lane_iota * 0
        magic = zeros16 + jnp.int32(0x5F3759DF)

        tvec = tidx_v[...]
        trow = []
        for k16 in range(n_sub):
            acc = ttab_v[pl.ds(k16 * _LANES, _LANES)]
            for j in range(1, 8):
                rowj = ttab_v[pl.ds(j * d_half + k16 * _LANES, _LANES)]
                acc = jnp.where(tvec == jnp.int32(j), rowj, acc)
            trow.append(acc)

        ts = trow[0]
        tsq = trow[0] * trow[0]
        for v in trow[1:]:
            ts = ts + v
            tsq = tsq + v * v
        tmu = _lane_sum(ts, lane_iota) * inv_n
        tvar = _lane_sum(tsq, lane_iota) * inv_n - tmu * tmu + _EPS
        tinv = _rsqrt_vec(tvar, magic)
        te = [(v - tmu) * tinv
              * tw_v[pl.ds(k16 * _LANES, _LANES)]
              + tb_v[pl.ds(k16 * _LANES, _LANES)]
              for k16, v in enumerate(trow)]

        dwv = [dw_v[pl.ds(k16 * _LANES, _LANES)] for k16 in range(n_sub)]
        dbv = [db_v[pl.ds(k16 * _LANES, _LANES)] for k16 in range(n_sub)]
        for r in range(stage_rows):
            xr = [raw_v[pl.ds(r * d_half + k16 * _LANES, _LANES)]
                  for k16 in range(n_sub)]
            s = xr[0]
            sq = xr[0] * xr[0]
            for v in xr[1:]:
                s = s + v
                sq = sq + v * v
            mu = _lane_sum(s, lane_iota) * inv_n
            var = _lane_sum(sq, lane_iota) * inv_n - mu * mu + _EPS
            inv = _rsqrt_vec(var, magic)
            for k16 in range(n_sub):
                comb_v[r, pl.ds(k16 * _LANES, _LANES)] = (
                    (xr[k16] - mu) * inv * dwv[k16] + dbv[k16])
                comb_v[r, pl.ds(d_half + k16 * _LANES, _LANES)] = te[k16]

        pltpu.sync_copy(comb_v, table_sp.at[pl.ds(start, stage_rows)])
        plsc.subcore_barrier()

        gathers = [
            pltpu.async_copy(table_sp.at[idx_v.at[j]], rows_v.at[j],
                             gsem.at[j])
            for j in range(n_chunks)
        ]
        stores = []
        for j in range(n_chunks):
            gathers[j].wait()
            stores.append(pltpu.async_copy(
                rows_v.at[j], out_hbm.at[base + j], ssem))
        for st in stores:
            st.wait()

    return k


def kernel(layer_indices, layer_type, depth_table, depth_ln_w, depth_ln_b,
           type_table, type_ln_w, type_ln_b):
    v, d = depth_table.shape
    b = layer_indices.shape[0]

    info = plsc.get_sparse_core_info()
    nc, ns = info.num_cores, info.num_subcores
    idx2d = layer_indices.astype(jnp.int32).reshape(-1, _IDX_CHUNK)
    tidx16 = jnp.full((_LANES,), jnp.asarray(layer_type, jnp.int32))
    out3d = _make_kernel(b, v, d, nc, ns)(
        depth_table.reshape(-1), idx2d, tidx16, depth_ln_w, depth_ln_b,
        type_table.reshape(-1), type_ln_w, type_ln_b)
    return out3d.reshape(b, 2 * d)


# speedup vs baseline: 1.2440x; 1.2440x over previous
import functools

import jax
import jax.numpy as jnp
from jax import lax
from jax.experimental import pallas as pl
from jax.experimental.pallas import tpu as pltpu
from jax.experimental.pallas import tpu_sc as plsc

_EPS = 1e-5
_IDX_CHUNK = 128


def _prep_body(depth_ref, dw_ref, db_ref, trow_ref, tw_ref, tb_ref, out_ref):
    x = depth_ref[...]
    mu = jnp.mean(x, axis=-1, keepdims=True)
    xc = x - mu
    var = jnp.mean(xc * xc, axis=-1, keepdims=True)
    d = xc * lax.rsqrt(var + _EPS) * dw_ref[...] + db_ref[...]

    t = trow_ref[...]
    tmu = jnp.mean(t, axis=-1, keepdims=True)
    tc = t - tmu
    tvar = jnp.mean(tc * tc, axis=-1, keepdims=True)
    te = tc * lax.rsqrt(tvar + _EPS) * tw_ref[...] + tb_ref[...]

    v = depth_ref.shape[0]
    out_ref[pl.ds(0, v), :] = jnp.concatenate(
        [d, jnp.broadcast_to(te, d.shape)], axis=-1)


def _make_gather(num_rows_total, table_rows_pad, row_width, nc, ns):
    nw = nc * ns
    rows_per_w = num_rows_total // nw
    n_chunks = rows_per_w // _IDX_CHUNK
    stage_rows = table_rows_pad // ns
    mesh = plsc.VectorSubcoreMesh(core_axis_name="c", subcore_axis_name="s")

    @functools.partial(
        pl.kernel,
        mesh=mesh,
        out_type=jax.ShapeDtypeStruct(
            (nw * n_chunks, _IDX_CHUNK, row_width), jnp.float32),
        scratch_types=[
            pltpu.VMEM_SHARED((table_rows_pad, row_width), jnp.float32),
            pltpu.VMEM((rows_per_w,), jnp.int32),
            pltpu.VMEM((n_chunks, _IDX_CHUNK, row_width), jnp.float32),
            pltpu.SemaphoreType.DMA((n_chunks,)),
            pltpu.SemaphoreType.DMA,
        ],
    )
    def gather_k(ctable_hbm, idx_hbm, out_hbm, table_sp, idx_v, rows_v,
                 gsem, ssem):
        sid = lax.axis_index("s")
        wid = sid * nc + lax.axis_index("c")
        base = wid * n_chunks
        pltpu.sync_copy(ctable_hbm.at[pl.ds(sid * stage_rows, stage_rows)],
                        table_sp.at[pl.ds(sid * stage_rows, stage_rows)])
        pltpu.sync_copy(idx_hbm.at[pl.ds(wid * rows_per_w, rows_per_w)],
                        idx_v)
        plsc.subcore_barrier()
        gathers = [
            pltpu.async_copy(
                table_sp.at[idx_v.at[pl.ds(j * _IDX_CHUNK, _IDX_CHUNK)]],
                rows_v.at[j], gsem.at[j])
            for j in range(n_chunks)
        ]
        stores = []
        for j in range(n_chunks):
            gathers[j].wait()
            stores.append(pltpu.async_copy(
                rows_v.at[j], out_hbm.at[base + j], ssem))
        for st in stores:
            st.wait()

    return gather_k


def kernel(layer_indices, layer_type, depth_table, depth_ln_w, depth_ln_b,
           type_table, type_ln_w, type_ln_b):
    v, d = depth_table.shape
    b = layer_indices.shape[0]
    v_pad = ((v + 127) // 128) * 128

    trow = lax.dynamic_slice_in_dim(
        type_table, jnp.asarray(layer_type, jnp.int32), 1, axis=0)
    ctable = pl.pallas_call(
        _prep_body,
        out_shape=jax.ShapeDtypeStruct((v_pad, 2 * d), jnp.float32),
    )(depth_table,
      depth_ln_w.reshape(1, d), depth_ln_b.reshape(1, d),
      trow, type_ln_w.reshape(1, d), type_ln_b.reshape(1, d))

    info = plsc.get_sparse_core_info()
    nc, ns = info.num_cores, info.num_subcores
    out3d = _make_gather(b, v_pad, 2 * d, nc, ns)(
        ctable, layer_indices.astype(jnp.int32))
    return out3d.reshape(b, 2 * d)


# speedup vs baseline: 1.2698x; 1.0207x over previous
import functools

import jax
import jax.numpy as jnp
from jax import lax
from jax.experimental import pallas as pl
from jax.experimental.pallas import tpu as pltpu
from jax.experimental.pallas import tpu_sc as plsc

_EPS = 1e-5
_IDX_CHUNK = 128


def _prep_body(depth_ref, dw_ref, db_ref, tidx_ref, ttab_ref, tw_ref, tb_ref,
               out_ref):
    x = depth_ref[...]
    mu = jnp.mean(x, axis=-1, keepdims=True)
    xc = x - mu
    var = jnp.mean(xc * xc, axis=-1, keepdims=True)
    d = xc * lax.rsqrt(var + _EPS) * dw_ref[...] + db_ref[...]

    tt = ttab_ref[...]
    row_ids = lax.broadcasted_iota(jnp.int32, tt.shape, 0)
    sel = jnp.where(row_ids == tidx_ref[...], tt, 0.0)
    t = jnp.sum(sel, axis=0, keepdims=True)
    tmu = jnp.mean(t, axis=-1, keepdims=True)
    tc = t - tmu
    tvar = jnp.mean(tc * tc, axis=-1, keepdims=True)
    te = tc * lax.rsqrt(tvar + _EPS) * tw_ref[...] + tb_ref[...]

    v = depth_ref.shape[0]
    out_ref[pl.ds(0, v), :] = jnp.concatenate(
        [d, jnp.broadcast_to(te, d.shape)], axis=-1)


def _make_gather(num_rows_total, table_rows_pad, row_width, nc, ns):
    nw = nc * ns
    rows_per_w = num_rows_total // nw
    n_chunks = rows_per_w // _IDX_CHUNK
    stage_rows = table_rows_pad // ns
    mesh = plsc.VectorSubcoreMesh(core_axis_name="c", subcore_axis_name="s")

    @functools.partial(
        pl.kernel,
        mesh=mesh,
        out_type=jax.ShapeDtypeStruct(
            (nw * n_chunks, _IDX_CHUNK, row_width), jnp.float32),
        scratch_types=[
            pltpu.VMEM_SHARED((table_rows_pad, row_width), jnp.float32),
            pltpu.VMEM((rows_per_w,), jnp.int32),
            pltpu.VMEM((n_chunks, _IDX_CHUNK, row_width), jnp.float32),
            pltpu.SemaphoreType.DMA((n_chunks,)),
            pltpu.SemaphoreType.DMA,
        ],
    )
    def gather_k(ctable_hbm, idx_hbm, out_hbm, table_sp, idx_v, rows_v,
                 gsem, ssem):
        sid = lax.axis_index("s")
        wid = sid * nc + lax.axis_index("c")
        base = wid * n_chunks
        pltpu.sync_copy(ctable_hbm.at[pl.ds(sid * stage_rows, stage_rows)],
                        table_sp.at[pl.ds(sid * stage_rows, stage_rows)])
        pltpu.sync_copy(idx_hbm.at[pl.ds(wid * rows_per_w, rows_per_w)],
                        idx_v)
        plsc.subcore_barrier()
        gathers = [
            pltpu.async_copy(
                table_sp.at[idx_v.at[pl.ds(j * _IDX_CHUNK, _IDX_CHUNK)]],
                rows_v.at[j], gsem.at[j])
            for j in range(n_chunks)
        ]
        stores = []
        for j in range(n_chunks):
            gathers[j].wait()
            stores.append(pltpu.async_copy(
                rows_v.at[j], out_hbm.at[base + j], ssem))
        for st in stores:
            st.wait()

    return gather_k


def kernel(layer_indices, layer_type, depth_table, depth_ln_w, depth_ln_b,
           type_table, type_ln_w, type_ln_b):
    v, d = depth_table.shape
    b = layer_indices.shape[0]
    v_pad = ((v + 127) // 128) * 128

    tidx = jnp.asarray(layer_type, jnp.int32).reshape(1, 1)
    ctable = pl.pallas_call(
        _prep_body,
        out_shape=jax.ShapeDtypeStruct((v_pad, 2 * d), jnp.float32),
    )(depth_table,
      depth_ln_w.reshape(1, d), depth_ln_b.reshape(1, d),
      tidx, type_table, type_ln_w.reshape(1, d), type_ln_b.reshape(1, d))

    info = plsc.get_sparse_core_info()
    nc, ns = info.num_cores, info.num_subcores
    out3d = _make_gather(b, v_pad, 2 * d, nc, ns)(
        ctable, layer_indices.astype(jnp.int32))
    return out3d.reshape(b, 2 * d)


# speedup vs baseline: 1.2862x; 1.0129x over previous
import functools

import jax
import jax.numpy as jnp
from jax import lax
from jax.experimental import pallas as pl
from jax.experimental.pallas import tpu as pltpu
from jax.experimental.pallas import tpu_sc as plsc

_EPS = 1e-5
_IDX_CHUNK = 64


def _prep_body(depth_ref, dw_ref, db_ref, tidx_ref, ttab_ref, tw_ref, tb_ref,
               out_ref):
    x = depth_ref[...]
    mu = jnp.mean(x, axis=-1, keepdims=True)
    xc = x - mu
    var = jnp.mean(xc * xc, axis=-1, keepdims=True)
    d = xc * lax.rsqrt(var + _EPS) * dw_ref[...] + db_ref[...]

    tt = ttab_ref[...]
    row_ids = lax.broadcasted_iota(jnp.int32, tt.shape, 0)
    sel = jnp.where(row_ids == tidx_ref[...], tt, 0.0)
    t = jnp.sum(sel, axis=0, keepdims=True)
    tmu = jnp.mean(t, axis=-1, keepdims=True)
    tc = t - tmu
    tvar = jnp.mean(tc * tc, axis=-1, keepdims=True)
    te = tc * lax.rsqrt(tvar + _EPS) * tw_ref[...] + tb_ref[...]

    v = depth_ref.shape[0]
    out_ref[pl.ds(0, v), :] = jnp.concatenate(
        [d, jnp.broadcast_to(te, d.shape)], axis=-1)


def _make_gather(num_rows_total, table_rows_pad, row_width, nc, ns):
    nw = nc * ns
    rows_per_w = num_rows_total // nw
    n_chunks = rows_per_w // _IDX_CHUNK
    stage_rows = table_rows_pad // ns
    mesh = plsc.VectorSubcoreMesh(core_axis_name="c", subcore_axis_name="s")

    @functools.partial(
        pl.kernel,
        mesh=mesh,
        out_type=jax.ShapeDtypeStruct(
            (nw * n_chunks, _IDX_CHUNK, row_width), jnp.float32),
        scratch_types=[
            pltpu.VMEM_SHARED((table_rows_pad, row_width), jnp.float32),
            pltpu.VMEM((rows_per_w,), jnp.int32),
            pltpu.VMEM((n_chunks, _IDX_CHUNK, row_width), jnp.float32),
            pltpu.SemaphoreType.DMA((n_chunks,)),
            pltpu.SemaphoreType.DMA,
        ],
    )
    def gather_k(ctable_hbm, idx_hbm, out_hbm, table_sp, idx_v, rows_v,
                 gsem, ssem):
        sid = lax.axis_index("s")
        wid = sid * nc + lax.axis_index("c")
        base = wid * n_chunks
        pltpu.sync_copy(ctable_hbm.at[pl.ds(sid * stage_rows, stage_rows)],
                        table_sp.at[pl.ds(sid * stage_rows, stage_rows)])
        pltpu.sync_copy(idx_hbm.at[pl.ds(wid * rows_per_w, rows_per_w)],
                        idx_v)
        plsc.subcore_barrier()
        gathers = [
            pltpu.async_copy(
                table_sp.at[idx_v.at[pl.ds(j * _IDX_CHUNK, _IDX_CHUNK)]],
                rows_v.at[j], gsem.at[j])
            for j in range(n_chunks)
        ]
        stores = []
        for j in range(n_chunks):
            gathers[j].wait()
            stores.append(pltpu.async_copy(
                rows_v.at[j], out_hbm.at[base + j], ssem))
        for st in stores:
            st.wait()

    return gather_k


def kernel(layer_indices, layer_type, depth_table, depth_ln_w, depth_ln_b,
           type_table, type_ln_w, type_ln_b):
    v, d = depth_table.shape
    b = layer_indices.shape[0]
    v_pad = ((v + 127) // 128) * 128

    tidx = jnp.asarray(layer_type, jnp.int32).reshape(1, 1)
    ctable = pl.pallas_call(
        _prep_body,
        out_shape=jax.ShapeDtypeStruct((v_pad, 2 * d), jnp.float32),
    )(depth_table,
      depth_ln_w.reshape(1, d), depth_ln_b.reshape(1, d),
      tidx, type_table, type_ln_w.reshape(1, d), type_ln_b.reshape(1, d))

    info = plsc.get_sparse_core_info()
    nc, ns = info.num_cores, info.num_subcores
    out3d = _make_gather(b, v_pad, 2 * d, nc, ns)(
        ctable, layer_indices.astype(jnp.int32))
    return out3d.reshape(b, 2 * d)


# speedup vs baseline: 1.2931x; 1.0053x over previous
import functools

import jax
import jax.numpy as jnp
from jax import lax
from jax.experimental import pallas as pl
from jax.experimental.pallas import tpu as pltpu
from jax.experimental.pallas import tpu_sc as plsc

_EPS = 1e-5
_IDX_CHUNK = 32


def _prep_body(depth_ref, dw_ref, db_ref, tidx_ref, ttab_ref, tw_ref, tb_ref,
               out_ref):
    x = depth_ref[...]
    mu = jnp.mean(x, axis=-1, keepdims=True)
    xc = x - mu
    var = jnp.mean(xc * xc, axis=-1, keepdims=True)
    d = xc * lax.rsqrt(var + _EPS) * dw_ref[...] + db_ref[...]

    tt = ttab_ref[...]
    row_ids = lax.broadcasted_iota(jnp.int32, tt.shape, 0)
    sel = jnp.where(row_ids == tidx_ref[...], tt, 0.0)
    t = jnp.sum(sel, axis=0, keepdims=True)
    tmu = jnp.mean(t, axis=-1, keepdims=True)
    tc = t - tmu
    tvar = jnp.mean(tc * tc, axis=-1, keepdims=True)
    te = tc * lax.rsqrt(tvar + _EPS) * tw_ref[...] + tb_ref[...]

    v = depth_ref.shape[0]
    out_ref[pl.ds(0, v), :] = jnp.concatenate(
        [d, jnp.broadcast_to(te, d.shape)], axis=-1)


def _make_gather(num_rows_total, table_rows_pad, row_width, nc, ns):
    nw = nc * ns
    rows_per_w = num_rows_total // nw
    n_chunks = rows_per_w // _IDX_CHUNK
    stage_rows = table_rows_pad // ns
    mesh = plsc.VectorSubcoreMesh(core_axis_name="c", subcore_axis_name="s")

    @functools.partial(
        pl.kernel,
        mesh=mesh,
        out_type=jax.ShapeDtypeStruct(
            (nw * n_chunks, _IDX_CHUNK, row_width), jnp.float32),
        scratch_types=[
            pltpu.VMEM_SHARED((table_rows_pad, row_width), jnp.float32),
            pltpu.VMEM((rows_per_w,), jnp.int32),
            pltpu.VMEM((n_chunks, _IDX_CHUNK, row_width), jnp.float32),
            pltpu.SemaphoreType.DMA((n_chunks,)),
            pltpu.SemaphoreType.DMA,
        ],
    )
    def gather_k(ctable_hbm, idx_hbm, out_hbm, table_sp, idx_v, rows_v,
                 gsem, ssem):
        sid = lax.axis_index("s")
        wid = sid * nc + lax.axis_index("c")
        base = wid * n_chunks
        pltpu.sync_copy(ctable_hbm.at[pl.ds(sid * stage_rows, stage_rows)],
                        table_sp.at[pl.ds(sid * stage_rows, stage_rows)])
        pltpu.sync_copy(idx_hbm.at[pl.ds(wid * rows_per_w, rows_per_w)],
                        idx_v)
        plsc.subcore_barrier()
        gathers = [
            pltpu.async_copy(
                table_sp.at[idx_v.at[pl.ds(j * _IDX_CHUNK, _IDX_CHUNK)]],
                rows_v.at[j], gsem.at[j])
            for j in range(n_chunks)
        ]
        stores = []
        for j in range(n_chunks):
            gathers[j].wait()
            stores.append(pltpu.async_copy(
                rows_v.at[j], out_hbm.at[base + j], ssem))
        for st in stores:
            st.wait()

    return gather_k


def kernel(layer_indices, layer_type, depth_table, depth_ln_w, depth_ln_b,
           type_table, type_ln_w, type_ln_b):
    v, d = depth_table.shape
    b = layer_indices.shape[0]
    v_pad = ((v + 127) // 128) * 128

    tidx = jnp.asarray(layer_type, jnp.int32).reshape(1, 1)
    ctable = pl.pallas_call(
        _prep_body,
        out_shape=jax.ShapeDtypeStruct((v_pad, 2 * d), jnp.float32),
    )(depth_table,
      depth_ln_w.reshape(1, d), depth_ln_b.reshape(1, d),
      tidx, type_table, type_ln_w.reshape(1, d), type_ln_b.reshape(1, d))

    info = plsc.get_sparse_core_info()
    nc, ns = info.num_cores, info.num_subcores
    out3d = _make_gather(b, v_pad, 2 * d, nc, ns)(
        ctable, layer_indices.astype(jnp.int32))
    return out3d.reshape(b, 2 * d)


# speedup vs baseline: 1.3185x; 1.0197x over previous
import functools

import jax
import jax.numpy as jnp
from jax import lax
from jax.experimental import pallas as pl
from jax.experimental.pallas import tpu as pltpu
from jax.experimental.pallas import tpu_sc as plsc

_EPS = 1e-5
_IDX_CHUNK = 32


def _prep_body(depth_ref, dw_ref, db_ref, tidx_ref, ttab_ref, tw_ref, tb_ref,
               out_ref):
    x = depth_ref[...]
    mu = jnp.mean(x, axis=-1, keepdims=True)
    xc = x - mu
    var = jnp.mean(xc * xc, axis=-1, keepdims=True)
    d = xc * lax.rsqrt(var + _EPS) * dw_ref[...] + db_ref[...]

    tt = ttab_ref[...]
    row_ids = lax.broadcasted_iota(jnp.int32, tt.shape, 0)
    sel = jnp.where(row_ids == tidx_ref[...], tt, 0.0)
    t = jnp.sum(sel, axis=0, keepdims=True)
    tmu = jnp.mean(t, axis=-1, keepdims=True)
    tc = t - tmu
    tvar = jnp.mean(tc * tc, axis=-1, keepdims=True)
    te = tc * lax.rsqrt(tvar + _EPS) * tw_ref[...] + tb_ref[...]

    v = depth_ref.shape[0]
    out_ref[pl.ds(0, v), :] = jnp.concatenate(
        [d, jnp.broadcast_to(te, d.shape)], axis=-1)


def _make_gather(num_rows_total, table_rows_pad, row_width, nc, ns):
    nw = nc * ns
    rows_per_w = num_rows_total // nw
    n_chunks = rows_per_w // _IDX_CHUNK
    stage_rows = table_rows_pad // ns
    mesh = plsc.VectorSubcoreMesh(core_axis_name="c", subcore_axis_name="s")

    @functools.partial(
        pl.kernel,
        mesh=mesh,
        out_type=jax.ShapeDtypeStruct(
            (nw * n_chunks, _IDX_CHUNK, row_width), jnp.float32),
        scratch_types=[
            pltpu.VMEM_SHARED((table_rows_pad, row_width), jnp.float32),
            pltpu.VMEM((rows_per_w,), jnp.int32),
            pltpu.VMEM((n_chunks, _IDX_CHUNK, row_width), jnp.float32),
            pltpu.SemaphoreType.DMA((n_chunks,)),
            pltpu.SemaphoreType.DMA,
        ],
    )
    def gather_k(ctable_hbm, idx_hbm, out_hbm, table_sp, idx_v, rows_v,
                 gsem, ssem):
        sid = lax.axis_index("s")
        wid = sid * nc + lax.axis_index("c")
        base = wid * n_chunks
        staging = pltpu.async_copy(
            ctable_hbm.at[pl.ds(sid * stage_rows, stage_rows)],
            table_sp.at[pl.ds(sid * stage_rows, stage_rows)], ssem)
        pltpu.sync_copy(idx_hbm.at[pl.ds(wid * rows_per_w, rows_per_w)],
                        idx_v)
        staging.wait()
        plsc.subcore_barrier()
        gathers = [
            pltpu.async_copy(
                table_sp.at[idx_v.at[pl.ds(j * _IDX_CHUNK, _IDX_CHUNK)]],
                rows_v.at[j], gsem.at[j])
            for j in range(n_chunks)
        ]
        stores = []
        for j in range(n_chunks):
            gathers[j].wait()
            stores.append(pltpu.async_copy(
                rows_v.at[j], out_hbm.at[base + j], ssem))
        for st in stores:
            st.wait()

    return gather_k


def kernel(layer_indices, layer_type, depth_table, depth_ln_w, depth_ln_b,
           type_table, type_ln_w, type_ln_b):
    v, d = depth_table.shape
    b = layer_indices.shape[0]
    v_pad = ((v + 127) // 128) * 128

    tidx = jnp.asarray(layer_type, jnp.int32).reshape(1, 1)
    ctable = pl.pallas_call(
        _prep_body,
        out_shape=jax.ShapeDtypeStruct((v_pad, 2 * d), jnp.float32),
    )(depth_table,
      depth_ln_w.reshape(1, d), depth_ln_b.reshape(1, d),
      tidx, type_table, type_ln_w.reshape(1, d), type_ln_b.reshape(1, d))

    info = plsc.get_sparse_core_info()
    nc, ns = info.num_cores, info.num_subcores
    out3d = _make_gather(b, v_pad, 2 * d, nc, ns)(
        ctable, layer_indices.astype(jnp.int32))
    return out3d.reshape(b, 2 * d)
